# baseline (speedup 1.0000x reference)
import functools

import jax
import jax.numpy as jnp
from jax import lax
from jax.experimental import pallas as pl
from jax.experimental.pallas import tpu as pltpu
from jax.experimental.pallas import tpu_sc as plsc

BATCH = 16384
D = 64
NC = 2
NS = 16
NW = NC * NS
BPW = BATCH // NW
CHUNK = 128
NCH = BPW // CHUNK

_mesh = plsc.VectorSubcoreMesh(core_axis_name="c", subcore_axis_name="s")


@functools.partial(
    pl.kernel,
    mesh=_mesh,
    out_type=jax.ShapeDtypeStruct((4, BATCH, 2 * D), jnp.int32),
    scratch_types=[
        pltpu.VMEM((NCH, CHUNK), jnp.int32),
        pltpu.VMEM((NCH, CHUNK), jnp.int32),
        pltpu.VMEM((BPW, 2 * D), jnp.int32),
        pltpu.SemaphoreType.DMA,
    ],
)
def _sc_gather(cat_i, col_i, fab_i, store_i, cat_t, col_t, fab_t, store_t,
               out, idx_v, qidx_v, rows_v, sem):
    wid = lax.axis_index("s") * NC + lax.axis_index("c")
    base = wid * NCH
    for t, (ih, th) in enumerate(
            [(cat_i, cat_t), (col_i, col_t), (fab_i, fab_t), (store_i, store_t)]):
        pltpu.sync_copy(ih.at[pl.ds(base, NCH)], idx_v)
        for j in range(NCH):
            for k in range(CHUNK // 16):
                qidx_v[j, pl.ds(16 * k, 16)] = lax.shift_right_logical(
                    idx_v[j, pl.ds(16 * k, 16)], 2)
        copies = []
        for j in range(NCH):
            copies.append(pltpu.async_copy(
                th.at[qidx_v.at[j]], rows_v.at[pl.ds(j * CHUNK, CHUNK)], sem))
        for c in copies:
            c.wait()
        pltpu.sync_copy(rows_v, out.at[t, pl.ds(wid * BPW, BPW)])


BLK = 1024
_HI_MASK = -65536


def _mm_body(e_ref, i_ref, wlo_ref, whi_ref, b_ref, o_ref):
    acc = jnp.broadcast_to(b_ref[...].astype(jnp.float32), (BLK, D))
    grp = lax.shift_right_logical(
        lax.broadcasted_iota(jnp.int32, (BLK, 2 * D), 1), 5)
    for t in range(4):
        q = i_ref[:, t:t + 1] & 3
        m = (grp == q).astype(jnp.float32)
        u = e_ref[t]
        lo = lax.bitcast_convert_type(u << 16, jnp.float32) * m
        hi = lax.bitcast_convert_type(u & _HI_MASK, jnp.float32) * m
        acc = acc + jnp.dot(lo, wlo_ref[t], preferred_element_type=jnp.float32)
        acc = acc + jnp.dot(hi, whi_ref[t], preferred_element_type=jnp.float32)
    o_ref[...] = acc


_mm = pl.pallas_call(
    _mm_body,
    grid=(BATCH // BLK,),
    in_specs=[
        pl.BlockSpec((4, BLK, 2 * D), lambda i: (0, i, 0)),
        pl.BlockSpec((BLK, 4), lambda i: (i, 0)),
        pl.BlockSpec((4, 2 * D, D), lambda i: (0, 0, 0)),
        pl.BlockSpec((4, 2 * D, D), lambda i: (0, 0, 0)),
        pl.BlockSpec((1, D), lambda i: (0, 0)),
    ],
    out_specs=pl.BlockSpec((BLK, D), lambda i: (i, 0)),
    out_shape=jax.ShapeDtypeStruct((BATCH, D), jnp.float32),
)


def _pack(table):
    b16 = table.astype(jnp.bfloat16)
    return lax.bitcast_convert_type(
        b16.reshape(-1, 2 * D, 2), jnp.int32)


def kernel(cat, col, fab, store, cat_table, col_table, fab_table, store_table, W, b):
    cat2 = cat.reshape(NW * NCH, CHUNK)
    col2 = col.reshape(NW * NCH, CHUNK)
    fab2 = fab.reshape(NW * NCH, CHUNK)
    store2 = store.reshape(NW * NCH, CHUNK)
    stage = _sc_gather(cat2, col2, fab2, store2,
                       _pack(cat_table), _pack(col_table),
                       _pack(fab_table), _pack(store_table))
    idx4 = jnp.stack([cat, col, fab, store], axis=1)
    wt = W.T.reshape(4, D, D)
    w2lo = jnp.tile(wt[:, 0::2, :], (1, 4, 1))
    w2hi = jnp.tile(wt[:, 1::2, :], (1, 4, 1))
    return _mm(stage, idx4, w2lo, w2hi, b.reshape(1, D))


# speedup vs baseline: 33.7643x; 33.7643x over previous
import functools

import jax
import jax.numpy as jnp
from jax import lax
from jax.experimental import pallas as pl
from jax.experimental.pallas import tpu as pltpu
from jax.experimental.pallas import tpu_sc as plsc

BATCH = 16384
D = 64
NC = 2
NS = 16
NW = NC * NS
BPW = BATCH // NW
NCAT = 1000
FCH = 16
GRP = BPW // 16

_mesh = plsc.VectorSubcoreMesh(core_axis_name="c", subcore_axis_name="s")


@functools.partial(
    pl.kernel,
    mesh=_mesh,
    out_type=(
        jax.ShapeDtypeStruct((2, NW, D, BPW), jnp.float32),
        jax.ShapeDtypeStruct((BATCH, 2 * D), jnp.float32),
        jax.ShapeDtypeStruct((BATCH, D), jnp.float32),
    ),
    scratch_types=[
        pltpu.VMEM((BPW,), jnp.int32),
        pltpu.VMEM((4, 128), jnp.int32),
        pltpu.VMEM((BPW // 2, 2 * D), jnp.float32),
        pltpu.VMEM((FCH, NCAT), jnp.float32),
        pltpu.VMEM((D, BPW), jnp.float32),
        pltpu.SemaphoreType.DMA,
        pltpu.SemaphoreType.DMA,
    ],
    compiler_params=pltpu.CompilerParams(needs_layout_passes=False),
)
def _sc_gather(cat_i, col_i, fab_i, store_i, catT, colT, fabP, store_t,
               cc_out, fab_out, store_out,
               idx_f, pidx_v, prow_v, tbl_v, ccT_v, sem_store, sem_fab):
    wid = lax.axis_index("s") * NC + lax.axis_index("c")
    base = wid * BPW

    pltpu.sync_copy(store_i.at[pl.ds(base, BPW)], idx_f)

    def srow(g, carry):
        v = idx_f[pl.ds(g * 16, 16)]
        for u in range(16):
            pltpu.make_async_copy(store_t.at[v[u]],
                                  store_out.at[base + g * 16 + u],
                                  sem_store).start()
        return carry

    lax.fori_loop(0, GRP, srow, None)

    pltpu.sync_copy(fab_i.at[pl.ds(base, BPW)], idx_f)
    for j in range(4):
        for k in range(8):
            pidx_v[j, pl.ds(16 * k, 16)] = lax.shift_right_logical(
                idx_f[pl.ds(j * 128 + 16 * k, 16)], 1)
    fcopies = [
        pltpu.async_copy(fabP.at[pidx_v.at[j]],
                         prow_v.at[pl.ds(j * 128, 128)], sem_fab)
        for j in range(2)
    ]

    for t, (ih, th) in enumerate([(cat_i, catT), (col_i, colT)]):
        pltpu.sync_copy(ih.at[pl.ds(base, BPW)], idx_f)
        for f0 in range(0, D, FCH):
            pltpu.sync_copy(th.at[pl.ds(f0, FCH)], tbl_v)

            def gbody(g, carry, f0=f0):
                rv = idx_f[pl.ds(g * 16, 16)]
                zv = lax.broadcasted_iota(jnp.int32, (16,), 0) * 0
                for fl in range(FCH):
                    ccT_v[f0 + fl, pl.ds(g * 16, 16)] = plsc.load_gather(
                        tbl_v, [zv + fl, rv])
                return carry

            lax.fori_loop(0, GRP, gbody, None)
        pltpu.sync_copy(ccT_v, cc_out.at[t, wid])
        if t == 0:
            for c in fcopies:
                c.wait()
            pltpu.sync_copy(prow_v, fab_out.at[pl.ds(base, BPW // 2)])
            fcopies = [
                pltpu.async_copy(fabP.at[pidx_v.at[j]],
                                 prow_v.at[pl.ds((j - 2) * 128, 128)], sem_fab)
                for j in range(2, 4)
            ]

    for c in fcopies:
        c.wait()
    pltpu.sync_copy(prow_v, fab_out.at[pl.ds(base + BPW // 2, BPW // 2)])
    pltpu.make_async_copy(store_t.at[pl.ds(0, BPW)],
                          store_out.at[pl.ds(base, BPW)], sem_store).wait()


BLK = BPW


def _mm_body(cc_ref, fab_ref, st_ref, i_ref, wt_ref, w2_ref, b_ref, o_ref):
    acc = jnp.broadcast_to(b_ref[...].astype(jnp.float32), (BLK, D))
    dnums = (((0,), (0,)), ((), ()))
    acc = acc + lax.dot_general(cc_ref[0, 0], wt_ref[0], dnums,
                                preferred_element_type=jnp.float32)
    acc = acc + lax.dot_general(cc_ref[1, 0], wt_ref[1], dnums,
                                preferred_element_type=jnp.float32)
    half = lax.broadcasted_iota(jnp.int32, (BLK, 2 * D), 1) >= D
    par = (i_ref[:, 0:1] & 1) == 1
    m = (half == par).astype(jnp.float32)
    acc = acc + jnp.dot(fab_ref[...] * m, w2_ref[...],
                        preferred_element_type=jnp.float32)
    acc = acc + jnp.dot(st_ref[...], wt_ref[3],
                        preferred_element_type=jnp.float32)
    o_ref[...] = acc


_mm = pl.pallas_call(
    _mm_body,
    grid=(BATCH // BLK,),
    in_specs=[
        pl.BlockSpec((2, 1, D, BLK), lambda i: (0, i, 0, 0)),
        pl.BlockSpec((BLK, 2 * D), lambda i: (i, 0)),
        pl.BlockSpec((BLK, D), lambda i: (i, 0)),
        pl.BlockSpec((BLK, 1), lambda i: (i, 0)),
        pl.BlockSpec((4, D, D), lambda i: (0, 0, 0)),
        pl.BlockSpec((2 * D, D), lambda i: (0, 0)),
        pl.BlockSpec((1, D), lambda i: (0, 0)),
    ],
    out_specs=pl.BlockSpec((BLK, D), lambda i: (i, 0)),
    out_shape=jax.ShapeDtypeStruct((BATCH, D), jnp.float32),
)


def kernel(cat, col, fab, store, cat_table, col_table, fab_table, store_table, W, b):
    cc, fabrows, strows = _sc_gather(
        cat, col, fab, store,
        cat_table.T, col_table.T,
        fab_table.reshape(-1, 2 * D), store_table)
    wt = W.T.reshape(4, D, D)
    w2 = jnp.concatenate([wt[2], wt[2]], axis=0)
    return _mm(cc, fabrows, strows, fab.reshape(BATCH, 1), wt, w2,
               b.reshape(1, D))


# speedup vs baseline: 35.0319x; 1.0375x over previous
import functools

import jax
import jax.numpy as jnp
from jax import lax
from jax.experimental import pallas as pl
from jax.experimental.pallas import tpu as pltpu
from jax.experimental.pallas import tpu_sc as plsc

BATCH = 16384
D = 64
NC = 2
NS = 16
NW = NC * NS
BPW = BATCH // NW
NCAT = 1000
FCH = 16
GRP = BPW // 16

_mesh = plsc.VectorSubcoreMesh(core_axis_name="c", subcore_axis_name="s")


@functools.partial(
    pl.kernel,
    mesh=_mesh,
    out_type=(
        jax.ShapeDtypeStruct((2, NW, D, BPW), jnp.float32),
        jax.ShapeDtypeStruct((BATCH, 2 * D), jnp.float32),
        jax.ShapeDtypeStruct((BATCH, D), jnp.float32),
    ),
    scratch_types=[
        pltpu.VMEM((BPW,), jnp.int32),
        pltpu.VMEM((4, 128), jnp.int32),
        pltpu.VMEM((BPW // 2, 2 * D), jnp.float32),
        pltpu.VMEM((FCH, NCAT), jnp.float32),
        pltpu.VMEM((D, BPW), jnp.float32),
        pltpu.SemaphoreType.DMA,
        pltpu.SemaphoreType.DMA,
    ],
    compiler_params=pltpu.CompilerParams(needs_layout_passes=False),
)
def _sc_gather(cat_i, col_i, fab_i, store_i, catT, colT, fabP, store_t,
               cc_out, fab_out, store_out,
               idx_f, pidx_v, prow_v, tbl_v, ccT_v, sem_store, sem_fab):
    wid = lax.axis_index("s") * NC + lax.axis_index("c")
    base = wid * BPW

    pltpu.sync_copy(store_i.at[pl.ds(base, BPW)], idx_f)

    def srow(g, carry):
        v = idx_f[pl.ds(g * 16, 16)]
        for u in range(16):
            pltpu.make_async_copy(store_t.at[v[u]],
                                  store_out.at[base + g * 16 + u],
                                  sem_store).start()
        return carry

    lax.fori_loop(0, GRP, srow, None)

    PROBE_SKIP_FAB = True
    PROBE_SKIP_CC = True
    pltpu.sync_copy(fab_i.at[pl.ds(base, BPW)], idx_f)
    for j in range(4):
        for k in range(8):
            pidx_v[j, pl.ds(16 * k, 16)] = lax.shift_right_logical(
                idx_f[pl.ds(j * 128 + 16 * k, 16)], 1)
    fcopies = [] if PROBE_SKIP_FAB else [
        pltpu.async_copy(fabP.at[pidx_v.at[j]],
                         prow_v.at[pl.ds(j * 128, 128)], sem_fab)
        for j in range(2)
    ]

    for t, (ih, th) in ([] if PROBE_SKIP_CC else
                        list(enumerate([(cat_i, catT), (col_i, colT)]))):
        pltpu.sync_copy(ih.at[pl.ds(base, BPW)], idx_f)
        for f0 in range(0, D, FCH):
            pltpu.sync_copy(th.at[pl.ds(f0, FCH)], tbl_v)

            def gbody(g, carry, f0=f0):
                rv = idx_f[pl.ds(g * 16, 16)]
                zv = lax.broadcasted_iota(jnp.int32, (16,), 0) * 0
                for fl in range(FCH):
                    ccT_v[f0 + fl, pl.ds(g * 16, 16)] = plsc.load_gather(
                        tbl_v, [zv + fl, rv])
                return carry

            lax.fori_loop(0, GRP, gbody, None)
        pltpu.sync_copy(ccT_v, cc_out.at[t, wid])
        if t == 0 and not PROBE_SKIP_FAB:
            for c in fcopies:
                c.wait()
            pltpu.sync_copy(prow_v, fab_out.at[pl.ds(base, BPW // 2)])
            fcopies = [
                pltpu.async_copy(fabP.at[pidx_v.at[j]],
                                 prow_v.at[pl.ds((j - 2) * 128, 128)], sem_fab)
                for j in range(2, 4)
            ]

    for c in fcopies:
        c.wait()
    if not PROBE_SKIP_FAB:
        pltpu.sync_copy(prow_v, fab_out.at[pl.ds(base + BPW // 2, BPW // 2)])
    pltpu.make_async_copy(store_t.at[pl.ds(0, BPW)],
                          store_out.at[pl.ds(base, BPW)], sem_store).wait()


BLK = BPW


def _mm_body(cc_ref, fab_ref, st_ref, i_ref, wt_ref, w2_ref, b_ref, o_ref):
    acc = jnp.broadcast_to(b_ref[...].astype(jnp.float32), (BLK, D))
    dnums = (((0,), (0,)), ((), ()))
    acc = acc + lax.dot_general(cc_ref[0, 0], wt_ref[0], dnums,
                                preferred_element_type=jnp.float32)
    acc = acc + lax.dot_general(cc_ref[1, 0], wt_ref[1], dnums,
                                preferred_element_type=jnp.float32)
    half = lax.broadcasted_iota(jnp.int32, (BLK, 2 * D), 1) >= D
    par = (i_ref[:, 0:1] & 1) == 1
    m = (half == par).astype(jnp.float32)
    acc = acc + jnp.dot(fab_ref[...] * m, w2_ref[...],
                        preferred_element_type=jnp.float32)
    acc = acc + jnp.dot(st_ref[...], wt_ref[3],
                        preferred_element_type=jnp.float32)
    o_ref[...] = acc


_mm = pl.pallas_call(
    _mm_body,
    grid=(BATCH // BLK,),
    in_specs=[
        pl.BlockSpec((2, 1, D, BLK), lambda i: (0, i, 0, 0)),
        pl.BlockSpec((BLK, 2 * D), lambda i: (i, 0)),
        pl.BlockSpec((BLK, D), lambda i: (i, 0)),
        pl.BlockSpec((BLK, 1), lambda i: (i, 0)),
        pl.BlockSpec((4, D, D), lambda i: (0, 0, 0)),
        pl.BlockSpec((2 * D, D), lambda i: (0, 0)),
        pl.BlockSpec((1, D), lambda i: (0, 0)),
    ],
    out_specs=pl.BlockSpec((BLK, D), lambda i: (i, 0)),
    out_shape=jax.ShapeDtypeStruct((BATCH, D), jnp.float32),
)


def kernel(cat, col, fab, store, cat_table, col_table, fab_table, store_table, W, b):
    cc, fabrows, strows = _sc_gather(
        cat, col, fab, store,
        cat_table.T, col_table.T,
        fab_table.reshape(-1, 2 * D), store_table)
    wt = W.T.reshape(4, D, D)
    w2 = jnp.concatenate([wt[2], wt[2]], axis=0)
    return _mm(cc, fabrows, strows, fab.reshape(BATCH, 1), wt, w2,
               b.reshape(1, D))


# speedup vs baseline: 48.1302x; 1.3739x over previous
import functools

import jax
import jax.numpy as jnp
from jax import lax
from jax.experimental import pallas as pl
from jax.experimental.pallas import tpu as pltpu
from jax.experimental.pallas import tpu_sc as plsc

BATCH = 16384
D = 64
NC = 2
NS = 16
NW = NC * NS
BPW = BATCH // NW
NCAT = 1000
FCH = 8
GRP = BPW // 16

_mesh = plsc.VectorSubcoreMesh(core_axis_name="c", subcore_axis_name="s")


@functools.partial(
    pl.kernel,
    mesh=_mesh,
    out_type=(
        jax.ShapeDtypeStruct((2, NW, D, BPW), jnp.float32),
        jax.ShapeDtypeStruct((BATCH, 2 * D), jnp.float32),
        jax.ShapeDtypeStruct((BATCH, D), jnp.float32),
    ),
    scratch_types=[
        pltpu.VMEM((BPW,), jnp.int32),
        pltpu.VMEM((4, 128), jnp.int32),
        pltpu.VMEM((BPW, D), jnp.float32),
        pltpu.VMEM((128, 2 * D), jnp.float32),
        pltpu.VMEM((FCH, NCAT), jnp.float32),
        pltpu.VMEM((D, BPW), jnp.float32),
        pltpu.SemaphoreType.DMA,
        pltpu.SemaphoreType.DMA,
    ],
    compiler_params=pltpu.CompilerParams(needs_layout_passes=False),
)
def _sc_gather(cat_i, col_i, fab_i, store_i, catT, colT, fabP, store_t,
               cc_out, fab_out, store_out,
               idx_f, pidx_v, rows_v, prow_v, tbl_v, ccT_v, sem_store, sem_fab):
    wid = lax.axis_index("s") * NC + lax.axis_index("c")
    base = wid * BPW

    pltpu.sync_copy(store_i.at[pl.ds(base, BPW)], idx_f)

    def srow(g, carry):
        v = idx_f[pl.ds(g * 16, 16)]
        for u in range(16):
            pltpu.make_async_copy(store_t.at[v[u]], rows_v.at[g * 16 + u],
                                  sem_store).start()
        return carry

    lax.fori_loop(0, GRP, srow, None)

    pltpu.sync_copy(fab_i.at[pl.ds(base, BPW)], idx_f)
    for j in range(4):
        for k in range(8):
            pidx_v[j, pl.ds(16 * k, 16)] = lax.shift_right_logical(
                idx_f[pl.ds(j * 128 + 16 * k, 16)], 1)
    for j in range(4):
        pltpu.async_copy(fabP.at[pidx_v.at[j]], prow_v, sem_fab).wait()
        pltpu.sync_copy(prow_v, fab_out.at[pl.ds(base + j * 128, 128)])

    for t, (ih, th) in enumerate([(cat_i, catT), (col_i, colT)]):
        pltpu.sync_copy(ih.at[pl.ds(base, BPW)], idx_f)
        for f0 in range(0, D, FCH):
            pltpu.sync_copy(th.at[pl.ds(f0, FCH)], tbl_v)

            def gbody(g, carry, f0=f0):
                rv = idx_f[pl.ds(g * 16, 16)]
                zv = lax.broadcasted_iota(jnp.int32, (16,), 0) * 0
                for fl in range(FCH):
                    ccT_v[f0 + fl, pl.ds(g * 16, 16)] = plsc.load_gather(
                        tbl_v, [zv + fl, rv])
                return carry

            lax.fori_loop(0, GRP, gbody, None)
        pltpu.sync_copy(ccT_v, cc_out.at[t, wid])

    pltpu.make_async_copy(store_t.at[pl.ds(0, BPW)], rows_v, sem_store).wait()
    pltpu.sync_copy(rows_v, store_out.at[pl.ds(base, BPW)])


BLK = BPW


def _mm_body(cc_ref, fab_ref, st_ref, i_ref, wt_ref, w2_ref, b_ref, o_ref):
    acc = jnp.broadcast_to(b_ref[...].astype(jnp.float32), (BLK, D))
    dnums = (((0,), (0,)), ((), ()))
    acc = acc + lax.dot_general(cc_ref[0, 0], wt_ref[0], dnums,
                                preferred_element_type=jnp.float32)
    acc = acc + lax.dot_general(cc_ref[1, 0], wt_ref[1], dnums,
                                preferred_element_type=jnp.float32)
    half = lax.broadcasted_iota(jnp.int32, (BLK, 2 * D), 1) >= D
    par = (i_ref[:, 0:1] & 1) == 1
    m = (half == par).astype(jnp.float32)
    acc = acc + jnp.dot(fab_ref[...] * m, w2_ref[...],
                        preferred_element_type=jnp.float32)
    acc = acc + jnp.dot(st_ref[...], wt_ref[3],
                        preferred_element_type=jnp.float32)
    o_ref[...] = acc


_mm = pl.pallas_call(
    _mm_body,
    grid=(BATCH // BLK,),
    in_specs=[
        pl.BlockSpec((2, 1, D, BLK), lambda i: (0, i, 0, 0)),
        pl.BlockSpec((BLK, 2 * D), lambda i: (i, 0)),
        pl.BlockSpec((BLK, D), lambda i: (i, 0)),
        pl.BlockSpec((BLK, 1), lambda i: (i, 0)),
        pl.BlockSpec((4, D, D), lambda i: (0, 0, 0)),
        pl.BlockSpec((2 * D, D), lambda i: (0, 0)),
        pl.BlockSpec((1, D), lambda i: (0, 0)),
    ],
    out_specs=pl.BlockSpec((BLK, D), lambda i: (i, 0)),
    out_shape=jax.ShapeDtypeStruct((BATCH, D), jnp.float32),
)


def kernel(cat, col, fab, store, cat_table, col_table, fab_table, store_table, W, b):
    cc, fabrows, strows = _sc_gather(
        cat, col, fab, store,
        cat_table.T, col_table.T,
        fab_table.reshape(-1, 2 * D), store_table)
    wt = W.T.reshape(4, D, D)
    w2 = jnp.concatenate([wt[2], wt[2]], axis=0)
    return _mm(cc, fabrows, strows, fab.reshape(BATCH, 1), wt, w2,
               b.reshape(1, D))


# speedup vs baseline: 48.4136x; 1.0059x over previous
import functools

import jax
import jax.numpy as jnp
from jax import lax
from jax.experimental import pallas as pl
from jax.experimental.pallas import tpu as pltpu
from jax.experimental.pallas import tpu_sc as plsc

BATCH = 16384
D = 64
NC = 2
NS = 16
NW = NC * NS
BPW = BATCH // NW
NCAT = 1000
FCH = 8
GRP = BPW // 16

_mesh = plsc.VectorSubcoreMesh(core_axis_name="c", subcore_axis_name="s")


@functools.partial(
    pl.kernel,
    mesh=_mesh,
    out_type=(
        jax.ShapeDtypeStruct((2, NW, D, BPW), jnp.float32),
        jax.ShapeDtypeStruct((BATCH, 2 * D), jnp.float32),
        jax.ShapeDtypeStruct((BATCH, D), jnp.float32),
    ),
    scratch_types=[
        pltpu.VMEM((BPW,), jnp.int32),
        pltpu.VMEM((4, 128), jnp.int32),
        pltpu.VMEM((BPW, D), jnp.float32),
        pltpu.VMEM((128, 2 * D), jnp.float32),
        pltpu.VMEM((FCH, NCAT), jnp.float32),
        pltpu.VMEM((D, BPW), jnp.float32),
        pltpu.SemaphoreType.DMA,
        pltpu.SemaphoreType.DMA,
    ],
    compiler_params=pltpu.CompilerParams(needs_layout_passes=False),
)
def _sc_gather(cat_i, col_i, fab_i, store_i, catT, colT, fabP, store_t,
               cc_out, fab_out, store_out,
               idx_f, pidx_v, rows_v, prow_v, tbl_v, ccT_v, sem_store, sem_fab):
    wid = lax.axis_index("s") * NC + lax.axis_index("c")
    base = wid * BPW

    pltpu.sync_copy(fab_i.at[pl.ds(base, BPW)], idx_f)
    for j in range(4):
        for k in range(8):
            pidx_v[j, pl.ds(16 * k, 16)] = lax.shift_right_logical(
                idx_f[pl.ds(j * 128 + 16 * k, 16)], 1)
    for j in range(4):
        pltpu.async_copy(fabP.at[pidx_v.at[j]], prow_v, sem_fab).wait()
        pltpu.sync_copy(prow_v, fab_out.at[pl.ds(base + j * 128, 128)])

    pltpu.sync_copy(store_i.at[pl.ds(base, BPW)], idx_f)

    def srow(g, carry):
        v = idx_f[pl.ds(g * 16, 16)]
        for u in range(16):
            pltpu.make_async_copy(store_t.at[v[u]], rows_v.at[g * 16 + u],
                                  sem_store).start()
        return carry

    lax.fori_loop(0, GRP, srow, None)

    for t, (ih, th) in enumerate([(cat_i, catT), (col_i, colT)]):
        pltpu.sync_copy(ih.at[pl.ds(base, BPW)], idx_f)
        for f0 in range(0, D, FCH):
            pltpu.sync_copy(th.at[pl.ds(f0, FCH)], tbl_v)

            def gbody(g, carry, f0=f0):
                rv = idx_f[pl.ds(g * 16, 16)]
                zv = lax.broadcasted_iota(jnp.int32, (16,), 0) * 0
                for fl in range(FCH):
                    ccT_v[f0 + fl, pl.ds(g * 16, 16)] = plsc.load_gather(
                        tbl_v, [zv + fl, rv])
                return carry

            lax.fori_loop(0, GRP, gbody, None)
        pltpu.sync_copy(ccT_v, cc_out.at[t, wid])

    pltpu.make_async_copy(store_t.at[pl.ds(0, BPW)], rows_v, sem_store).wait()
    pltpu.sync_copy(rows_v, store_out.at[pl.ds(base, BPW)])


BLK = BPW


def _mm_body(cc_ref, fab_ref, st_ref, i_ref, wt_ref, w2_ref, b_ref, o_ref):
    acc = jnp.broadcast_to(b_ref[...].astype(jnp.float32), (BLK, D))
    dnums = (((0,), (0,)), ((), ()))
    acc = acc + lax.dot_general(cc_ref[0, 0], wt_ref[0], dnums,
                                preferred_element_type=jnp.float32)
    acc = acc + lax.dot_general(cc_ref[1, 0], wt_ref[1], dnums,
                                preferred_element_type=jnp.float32)
    half = lax.broadcasted_iota(jnp.int32, (BLK, 2 * D), 1) >= D
    par = (i_ref[:, 0:1] & 1) == 1
    m = (half == par).astype(jnp.float32)
    acc = acc + jnp.dot(fab_ref[...] * m, w2_ref[...],
                        preferred_element_type=jnp.float32)
    acc = acc + jnp.dot(st_ref[...], wt_ref[3],
                        preferred_element_type=jnp.float32)
    o_ref[...] = acc


_mm = pl.pallas_call(
    _mm_body,
    grid=(BATCH // BLK,),
    in_specs=[
        pl.BlockSpec((2, 1, D, BLK), lambda i: (0, i, 0, 0)),
        pl.BlockSpec((BLK, 2 * D), lambda i: (i, 0)),
        pl.BlockSpec((BLK, D), lambda i: (i, 0)),
        pl.BlockSpec((BLK, 1), lambda i: (i, 0)),
        pl.BlockSpec((4, D, D), lambda i: (0, 0, 0)),
        pl.BlockSpec((2 * D, D), lambda i: (0, 0)),
        pl.BlockSpec((1, D), lambda i: (0, 0)),
    ],
    out_specs=pl.BlockSpec((BLK, D), lambda i: (i, 0)),
    out_shape=jax.ShapeDtypeStruct((BATCH, D), jnp.float32),
)


def kernel(cat, col, fab, store, cat_table, col_table, fab_table, store_table, W, b):
    cc, fabrows, strows = _sc_gather(
        cat, col, fab, store,
        cat_table.T, col_table.T,
        fab_table.reshape(-1, 2 * D), store_table)
    wt = W.T.reshape(4, D, D)
    w2 = jnp.concatenate([wt[2], wt[2]], axis=0)
    return _mm(cc, fabrows, strows, fab.reshape(BATCH, 1), wt, w2,
               b.reshape(1, D))


# speedup vs baseline: 49.2093x; 1.0164x over previous
import functools

import jax
import jax.numpy as jnp
from jax import lax
from jax.experimental import pallas as pl
from jax.experimental.pallas import tpu as pltpu
from jax.experimental.pallas import tpu_sc as plsc

BATCH = 16384
D = 64
NC = 2
NS = 16
NW = NC * NS
BPW = BATCH // NW
NCAT = 1000
FCH = 8
GRP = BPW // 16

_mesh = plsc.VectorSubcoreMesh(core_axis_name="c", subcore_axis_name="s")


@functools.partial(
    pl.kernel,
    mesh=_mesh,
    out_type=(
        jax.ShapeDtypeStruct((2, NW, D, BPW), jnp.float32),
        jax.ShapeDtypeStruct((BATCH, 2 * D), jnp.float32),
        jax.ShapeDtypeStruct((BATCH, D), jnp.float32),
    ),
    scratch_types=[
        pltpu.VMEM((BPW,), jnp.int32),
        pltpu.VMEM((4, 128), jnp.int32),
        pltpu.VMEM((BPW, D), jnp.float32),
        pltpu.VMEM((128, 2 * D), jnp.float32),
        pltpu.VMEM((FCH, NCAT), jnp.float32),
        pltpu.VMEM((D, BPW), jnp.float32),
        pltpu.SemaphoreType.DMA,
        pltpu.SemaphoreType.DMA,
    ],
    compiler_params=pltpu.CompilerParams(needs_layout_passes=False),
)
def _sc_gather(cat_i, col_i, fab_i, store_i, catT, colT, fabP, store_t,
               cc_out, fab_out, store_out,
               idx_f, pidx_v, rows_v, prow_v, tbl_v, ccT_v, sem_store, sem_fab):
    wid = lax.axis_index("s") * NC + lax.axis_index("c")
    base = wid * BPW

    pltpu.sync_copy(fab_i.at[pl.ds(base, BPW)], idx_f)
    for j in range(4):
        for k in range(8):
            pidx_v[j, pl.ds(16 * k, 16)] = lax.shift_right_logical(
                idx_f[pl.ds(j * 128 + 16 * k, 16)], 1)
    for j in range(0):
        pltpu.async_copy(fabP.at[pidx_v.at[j]], prow_v, sem_fab).wait()
        pltpu.sync_copy(prow_v, fab_out.at[pl.ds(base + j * 128, 128)])

    pltpu.sync_copy(store_i.at[pl.ds(base, BPW)], idx_f)

    def srow(g, carry):
        v = idx_f[pl.ds(g * 16, 16)]
        for u in range(16):
            pltpu.make_async_copy(store_t.at[v[u]], rows_v.at[g * 16 + u],
                                  sem_store).start()
        return carry

    lax.fori_loop(0, 0, srow, None)

    for t, (ih, th) in enumerate([(cat_i, catT), (col_i, colT)]):
        pltpu.sync_copy(ih.at[pl.ds(base, BPW)], idx_f)
        for f0 in range(0, D, FCH):
            pltpu.sync_copy(th.at[pl.ds(f0, FCH)], tbl_v)

            def gbody(g, carry, f0=f0):
                rv = idx_f[pl.ds(g * 16, 16)]
                zv = lax.broadcasted_iota(jnp.int32, (16,), 0) * 0
                for fl in range(FCH):
                    ccT_v[f0 + fl, pl.ds(g * 16, 16)] = plsc.load_gather(
                        tbl_v, [zv + fl, rv])
                return carry

            lax.fori_loop(0, GRP, gbody, None)
        pltpu.sync_copy(ccT_v, cc_out.at[t, wid])

    pltpu.sync_copy(rows_v, store_out.at[pl.ds(base, BPW)])


BLK = BPW


def _mm_body(cc_ref, fab_ref, st_ref, i_ref, wt_ref, w2_ref, b_ref, o_ref):
    acc = jnp.broadcast_to(b_ref[...].astype(jnp.float32), (BLK, D))
    dnums = (((0,), (0,)), ((), ()))
    acc = acc + lax.dot_general(cc_ref[0, 0], wt_ref[0], dnums,
                                preferred_element_type=jnp.float32)
    acc = acc + lax.dot_general(cc_ref[1, 0], wt_ref[1], dnums,
                                preferred_element_type=jnp.float32)
    half = lax.broadcasted_iota(jnp.int32, (BLK, 2 * D), 1) >= D
    par = (i_ref[:, 0:1] & 1) == 1
    m = (half == par).astype(jnp.float32)
    acc = acc + jnp.dot(fab_ref[...] * m, w2_ref[...],
                        preferred_element_type=jnp.float32)
    acc = acc + jnp.dot(st_ref[...], wt_ref[3],
                        preferred_element_type=jnp.float32)
    o_ref[...] = acc


_mm = pl.pallas_call(
    _mm_body,
    grid=(BATCH // BLK,),
    in_specs=[
        pl.BlockSpec((2, 1, D, BLK), lambda i: (0, i, 0, 0)),
        pl.BlockSpec((BLK, 2 * D), lambda i: (i, 0)),
        pl.BlockSpec((BLK, D), lambda i: (i, 0)),
        pl.BlockSpec((BLK, 1), lambda i: (i, 0)),
        pl.BlockSpec((4, D, D), lambda i: (0, 0, 0)),
        pl.BlockSpec((2 * D, D), lambda i: (0, 0)),
        pl.BlockSpec((1, D), lambda i: (0, 0)),
    ],
    out_specs=pl.BlockSpec((BLK, D), lambda i: (i, 0)),
    out_shape=jax.ShapeDtypeStruct((BATCH, D), jnp.float32),
)


def kernel(cat, col, fab, store, cat_table, col_table, fab_table, store_table, W, b):
    cc, fabrows, strows = _sc_gather(
        cat, col, fab, store,
        cat_table.T, col_table.T,
        fab_table.reshape(-1, 2 * D), store_table)
    wt = W.T.reshape(4, D, D)
    w2 = jnp.concatenate([wt[2], wt[2]], axis=0)
    return _mm(cc, fabrows, strows, fab.reshape(BATCH, 1), wt, w2,
               b.reshape(1, D))


# speedup vs baseline: 51.0505x; 1.0374x over previous
import functools

import jax
import jax.numpy as jnp
from jax import lax
from jax.experimental import pallas as pl
from jax.experimental.pallas import tpu as pltpu
from jax.experimental.pallas import tpu_sc as plsc

BATCH = 16384
D = 64
NC = 2
NS = 16
NW = NC * NS
BPW = BATCH // NW
GRP = BPW // 16

_mesh = plsc.VectorSubcoreMesh(core_axis_name="c", subcore_axis_name="s")


@functools.partial(
    pl.kernel,
    mesh=_mesh,
    out_type=(
        jax.ShapeDtypeStruct((3, BATCH, 2 * D), jnp.float32),
        jax.ShapeDtypeStruct((BATCH, D), jnp.float32),
    ),
    scratch_types=[
        pltpu.VMEM((BPW,), jnp.int32),
        pltpu.VMEM((4, 128), jnp.int32),
        pltpu.VMEM((BPW, D), jnp.float32),
        pltpu.VMEM((BPW // 2, 2 * D), jnp.float32),
        pltpu.SemaphoreType.DMA,
        pltpu.SemaphoreType.DMA,
    ],
    compiler_params=pltpu.CompilerParams(needs_layout_passes=False),
)
def _sc_gather(cat_i, col_i, fab_i, store_i, catP, colP, fabP, store_t,
               pair_out, store_out,
               idx_f, pidx_v, rows_v, prow_v, sem_store, sem_pair):
    wid = lax.axis_index("s") * NC + lax.axis_index("c")
    base = wid * BPW

    pltpu.sync_copy(store_i.at[pl.ds(base, BPW)], idx_f)

    def srow(g, carry):
        v = idx_f[pl.ds(g * 16, 16)]
        for u in range(16):
            pltpu.make_async_copy(store_t.at[v[u]], rows_v.at[g * 16 + u],
                                  sem_store).start()
        return carry

    lax.fori_loop(0, GRP, srow, None)

    for t, (ih, th) in enumerate([(cat_i, catP), (col_i, colP), (fab_i, fabP)]):
        pltpu.sync_copy(ih.at[pl.ds(base, BPW)], idx_f)
        for j in range(4):
            for k in range(8):
                pidx_v[j, pl.ds(16 * k, 16)] = lax.shift_right_logical(
                    idx_f[pl.ds(j * 128 + 16 * k, 16)], 1)
        for h in range(2):
            copies = [
                pltpu.async_copy(th.at[pidx_v.at[2 * h + j]],
                                 prow_v.at[pl.ds(j * 128, 128)], sem_pair)
                for j in range(2)
            ]
            for c in copies:
                c.wait()
            pltpu.sync_copy(
                prow_v, pair_out.at[t, pl.ds(base + h * 256, 256)])

    pltpu.make_async_copy(store_t.at[pl.ds(0, BPW)], rows_v, sem_store).wait()
    pltpu.sync_copy(rows_v, store_out.at[pl.ds(base, BPW)])


BLK = 1024


def _mm_body(e_ref, st_ref, i_ref, w2_ref, wt3_ref, b_ref, o_ref):
    acc = jnp.broadcast_to(b_ref[...].astype(jnp.float32), (BLK, D))
    half = lax.broadcasted_iota(jnp.int32, (BLK, 2 * D), 1) >= D
    for t in range(3):
        par = (i_ref[:, t:t + 1] & 1) == 1
        m = (half == par).astype(jnp.float32)
        acc = acc + jnp.dot(e_ref[t] * m, w2_ref[t],
                            preferred_element_type=jnp.float32)
    acc = acc + jnp.dot(st_ref[...], wt3_ref[...],
                        preferred_element_type=jnp.float32)
    o_ref[...] = acc


_mm = pl.pallas_call(
    _mm_body,
    grid=(BATCH // BLK,),
    in_specs=[
        pl.BlockSpec((3, BLK, 2 * D), lambda i: (0, i, 0)),
        pl.BlockSpec((BLK, D), lambda i: (i, 0)),
        pl.BlockSpec((BLK, 4), lambda i: (i, 0)),
        pl.BlockSpec((3, 2 * D, D), lambda i: (0, 0, 0)),
        pl.BlockSpec((D, D), lambda i: (0, 0)),
        pl.BlockSpec((1, D), lambda i: (0, 0)),
    ],
    out_specs=pl.BlockSpec((BLK, D), lambda i: (i, 0)),
    out_shape=jax.ShapeDtypeStruct((BATCH, D), jnp.float32),
)


def kernel(cat, col, fab, store, cat_table, col_table, fab_table, store_table, W, b):
    pairs, strows = _sc_gather(
        cat, col, fab, store,
        cat_table.reshape(-1, 2 * D),
        col_table.reshape(-1, 2 * D),
        fab_table.reshape(-1, 2 * D),
        store_table)
    idx4 = jnp.stack([cat, col, fab, store], axis=1)
    wt = W.T.reshape(4, D, D)
    w2 = jnp.concatenate([wt[:3], wt[:3]], axis=1)
    return _mm(pairs, strows, idx4, w2, wt[3], b.reshape(1, D))


# speedup vs baseline: 54.6924x; 1.0713x over previous
import functools

import jax
import jax.numpy as jnp
from jax import lax
from jax.experimental import pallas as pl
from jax.experimental.pallas import tpu as pltpu
from jax.experimental.pallas import tpu_sc as plsc

BATCH = 16384
D = 64
NC = 2
NS = 16
NW = NC * NS
BPW = BATCH // NW
UNROLL = 16

_mesh = plsc.VectorSubcoreMesh(core_axis_name="c", subcore_axis_name="s")


@functools.partial(
    pl.kernel,
    mesh=_mesh,
    out_type=jax.ShapeDtypeStruct((4, BATCH, D), jnp.float32),
    scratch_types=[
        pltpu.VMEM((BPW,), jnp.int32),
        pltpu.VMEM((BPW, D), jnp.float32),
        pltpu.SemaphoreType.DMA,
    ],
)
def _sc_gather(cat_i, col_i, fab_i, store_i, cat_t, col_t, fab_t, store_t,
               out, idx_v, rows_v, sem):
    wid = lax.axis_index("s") * NC + lax.axis_index("c")
    base = wid * BPW
    for t, (ih, th) in enumerate(
            [(cat_i, cat_t), (col_i, col_t), (fab_i, fab_t), (store_i, store_t)]):
        pltpu.sync_copy(ih.at[pl.ds(base, BPW)], idx_v)

        def body(g, _, th=th):
            v = idx_v[pl.ds(g * UNROLL, UNROLL)]
            for u in range(UNROLL):
                r = v[u]
                pltpu.make_async_copy(th.at[r], rows_v.at[g * UNROLL + u],
                                      sem).start()
            return _

        lax.fori_loop(0, BPW // UNROLL, body, None)
        pltpu.make_async_copy(th.at[pl.ds(0, BPW)], rows_v, sem).wait()
        pltpu.sync_copy(rows_v, out.at[t, pl.ds(base, BPW)])


BLK = 1024


def _mm_body(e_ref, wt_ref, b_ref, o_ref):
    acc = jnp.broadcast_to(b_ref[...].astype(jnp.float32), (BLK, D))
    for t in range(4):
        acc = acc + jnp.dot(e_ref[t], wt_ref[t],
                            preferred_element_type=jnp.float32)
    o_ref[...] = acc


_mm = pl.pallas_call(
    _mm_body,
    grid=(BATCH // BLK,),
    in_specs=[
        pl.BlockSpec((4, BLK, D), lambda i: (0, i, 0)),
        pl.BlockSpec((4, D, D), lambda i: (0, 0, 0)),
        pl.BlockSpec((1, D), lambda i: (0, 0)),
    ],
    out_specs=pl.BlockSpec((BLK, D), lambda i: (i, 0)),
    out_shape=jax.ShapeDtypeStruct((BATCH, D), jnp.float32),
)


def kernel(cat, col, fab, store, cat_table, col_table, fab_table, store_table, W, b):
    stage = _sc_gather(cat, col, fab, store,
                       cat_table, col_table, fab_table, store_table)
    wt = W.T.reshape(4, D, D)
    return _mm(stage, wt, b.reshape(1, D))
